# baseline (speedup 1.0000x reference)
import jax
import jax.numpy as jnp
from jax import lax
from jax.experimental import pallas as pl
from jax.experimental.pallas import tpu as pltpu
from jax.experimental.pallas import tpu_sc as plsc

B = 16384
D = 64
H = 64
NC = 2
NS = 16
NW = NC * NS
BPW = B // NW
CHUNK = 128
NCH = BPW // CHUNK

TBLK = 1024
NTS = 489
NPAIR = NTS * TBLK


def _transpose_body(lo_ref, hi_ref, o_ref):
    eye = jnp.eye(D, dtype=jnp.float32)
    dnT = (((0,), (0,)), ((), ()))
    o_ref[:, :D] = lax.dot_general(lo_ref[...], eye, dnT,
                                   preferred_element_type=jnp.float32,
                                   precision=lax.Precision.HIGHEST)
    o_ref[:, D:] = lax.dot_general(hi_ref[...], eye, dnT,
                                   preferred_element_type=jnp.float32,
                                   precision=lax.Precision.HIGHEST)


def _pair_transpose(tT):
    return pl.pallas_call(
        _transpose_body,
        grid=(NTS,),
        in_specs=[pl.BlockSpec((D, TBLK), lambda i: (0, i)),
                  pl.BlockSpec((D, TBLK),
                               lambda i: (0, jnp.minimum(i + NTS, 976)))],
        out_specs=pl.BlockSpec((TBLK, 2 * D), lambda i: (i, 0)),
        out_shape=jax.ShapeDtypeStruct((NPAIR, 2 * D), jnp.float32),
    )(tT, tT)


def _gather_body(u_tbl, i_tbl, u_idx, i_idx, u_out, i_out,
                 idx_v, rows_v, sem):
    wid = lax.axis_index("s") * NC + lax.axis_index("c")
    base = wid * BPW

    pltpu.sync_copy(u_idx.at[pl.ds(base, BPW)], idx_v)
    copies = [
        pltpu.async_copy(u_tbl.at[idx_v.at[pl.ds(c * CHUNK, CHUNK)]],
                         rows_v.at[pl.ds(c * CHUNK, CHUNK)], sem)
        for c in range(NCH)
    ]
    for cp in copies:
        cp.wait()
    pltpu.sync_copy(rows_v, u_out.at[pl.ds(base, BPW)])

    pltpu.sync_copy(i_idx.at[pl.ds(base, BPW)], idx_v)
    copies = [
        pltpu.async_copy(i_tbl.at[idx_v.at[pl.ds(c * CHUNK, CHUNK)]],
                         rows_v.at[pl.ds(c * CHUNK, CHUNK)], sem)
        for c in range(NCH)
    ]
    for cp in copies:
        cp.wait()
    pltpu.sync_copy(rows_v, i_out.at[pl.ds(base, BPW)])


def _sc_gather(user_pairs, item_pairs, user_idx, item_idx):
    mesh = plsc.VectorSubcoreMesh(core_axis_name="c", subcore_axis_name="s")
    kern = pl.kernel(
        _gather_body,
        out_type=[jax.ShapeDtypeStruct((B, 2 * D), jnp.float32),
                  jax.ShapeDtypeStruct((B, 2 * D), jnp.float32)],
        mesh=mesh,
        scratch_types=[
            pltpu.VMEM((BPW,), jnp.int32),
            pltpu.VMEM((BPW, 2 * D), jnp.float32),
            pltpu.SemaphoreType.DMA,
        ],
    )
    return kern(user_pairs, item_pairs, user_idx, item_idx)


def _mlp_body(gu_ref, gi_ref, pu_ref, pi_ref, w1_ref, b1_ref, w2_ref, b2_ref,
              o_ref):
    ones_row = jnp.ones((1, D), jnp.float32)
    dn0 = (((0,), (0,)), ((), ()))
    pu = lax.dot_general(pu_ref[...], ones_row, dn0,
                         preferred_element_type=jnp.float32)
    pi = lax.dot_general(pi_ref[...], ones_row, dn0,
                         preferred_element_type=jnp.float32)
    gu = gu_ref[...]
    gi = gi_ref[...]
    uv = gu[:, :D] + pu * (gu[:, D:] - gu[:, :D])
    iv = gi[:, :D] + pi * (gi[:, D:] - gi[:, :D])
    w1 = w1_ref[...]
    dn1 = (((1,), (1,)), ((), ()))
    h = lax.dot_general(uv, w1[:, :D], dn1,
                        preferred_element_type=jnp.float32,
                        precision=lax.Precision.HIGHEST)
    h = h + lax.dot_general(iv, w1[:, D:], dn1,
                            preferred_element_type=jnp.float32,
                            precision=lax.Precision.HIGHEST)
    h = jnp.maximum(h + b1_ref[...], 0.0)
    o = jnp.sum(h * w2_ref[...], axis=1, keepdims=True)
    o_ref[...] = jax.nn.sigmoid(o + b2_ref[0, 0])


def kernel(user_indices, item_indices, user_table, item_table, W1, b1, W2, b2):
    u32 = user_indices.astype(jnp.int32)
    i32 = item_indices.astype(jnp.int32)
    up = _pair_transpose(user_table.T)
    ip = _pair_transpose(item_table.T)
    u_hi = (u32 >= NPAIR).astype(jnp.int32)
    i_hi = (i32 >= NPAIR).astype(jnp.int32)
    gu, gi = _sc_gather(up, ip, u32 - u_hi * NPAIR, i32 - i_hi * NPAIR)
    pu = u_hi.astype(jnp.float32).reshape(1, B)
    pi = i_hi.astype(jnp.float32).reshape(1, B)
    blk = 1024
    out = pl.pallas_call(
        _mlp_body,
        grid=(B // blk,),
        in_specs=[
            pl.BlockSpec((blk, 2 * D), lambda i: (i, 0)),
            pl.BlockSpec((blk, 2 * D), lambda i: (i, 0)),
            pl.BlockSpec((1, blk), lambda i: (0, i)),
            pl.BlockSpec((1, blk), lambda i: (0, i)),
            pl.BlockSpec((H, 2 * D), lambda i: (0, 0)),
            pl.BlockSpec((1, H), lambda i: (0, 0)),
            pl.BlockSpec((1, H), lambda i: (0, 0)),
            pl.BlockSpec((1, 1), lambda i: (0, 0)),
        ],
        out_specs=pl.BlockSpec((blk, 1), lambda i: (i, 0)),
        out_shape=jax.ShapeDtypeStruct((B, 1), jnp.float32),
    )(gu, gi, pu, pi, W1, b1.reshape(1, H), W2, b2.reshape(1, 1))
    return out.reshape(B)


# speedup vs baseline: 1.3242x; 1.3242x over previous
import jax
import jax.numpy as jnp
from jax import lax
from jax.experimental import pallas as pl
from jax.experimental.pallas import tpu as pltpu
from jax.experimental.pallas import tpu_sc as plsc

B = 16384
D = 64
H = 64
NC = 2
NS = 16
NW = NC * NS
BPW = B // NW
CHUNK = 128
NCH = BPW // CHUNK

TBLK = 1024
NTS = 489
NPAIR = NTS * TBLK


def _transpose_body(lo_ref, hi_ref, o_ref):
    eye = jnp.eye(D, dtype=jnp.float32)
    dnT = (((0,), (0,)), ((), ()))
    o_ref[:, :D] = lax.dot_general(lo_ref[...], eye, dnT,
                                   preferred_element_type=jnp.float32)
    o_ref[:, D:] = lax.dot_general(hi_ref[...], eye, dnT,
                                   preferred_element_type=jnp.float32)


def _pair_transpose(tT):
    return pl.pallas_call(
        _transpose_body,
        grid=(NTS,),
        in_specs=[pl.BlockSpec((D, TBLK), lambda i: (0, i)),
                  pl.BlockSpec((D, TBLK),
                               lambda i: (0, jnp.minimum(i + NTS, 976)))],
        out_specs=pl.BlockSpec((TBLK, 2 * D), lambda i: (i, 0)),
        out_shape=jax.ShapeDtypeStruct((NPAIR, 2 * D), jnp.float32),
    )(tT, tT)


def _gather_body(u_tbl, i_tbl, u_idx, i_idx, u_out, i_out,
                 idx_v, rows_v, sem):
    wid = lax.axis_index("s") * NC + lax.axis_index("c")
    base = wid * BPW

    pltpu.sync_copy(u_idx.at[pl.ds(base, BPW)], idx_v)
    copies = [
        pltpu.async_copy(u_tbl.at[idx_v.at[pl.ds(c * CHUNK, CHUNK)]],
                         rows_v.at[pl.ds(c * CHUNK, CHUNK)], sem)
        for c in range(NCH)
    ]
    for cp in copies:
        cp.wait()
    pltpu.sync_copy(rows_v, u_out.at[pl.ds(base, BPW)])

    pltpu.sync_copy(i_idx.at[pl.ds(base, BPW)], idx_v)
    copies = [
        pltpu.async_copy(i_tbl.at[idx_v.at[pl.ds(c * CHUNK, CHUNK)]],
                         rows_v.at[pl.ds(c * CHUNK, CHUNK)], sem)
        for c in range(NCH)
    ]
    for cp in copies:
        cp.wait()
    pltpu.sync_copy(rows_v, i_out.at[pl.ds(base, BPW)])


def _sc_gather(user_pairs, item_pairs, user_idx, item_idx):
    mesh = plsc.VectorSubcoreMesh(core_axis_name="c", subcore_axis_name="s")
    kern = pl.kernel(
        _gather_body,
        out_type=[jax.ShapeDtypeStruct((B, 2 * D), jnp.float32),
                  jax.ShapeDtypeStruct((B, 2 * D), jnp.float32)],
        mesh=mesh,
        scratch_types=[
            pltpu.VMEM((BPW,), jnp.int32),
            pltpu.VMEM((BPW, 2 * D), jnp.float32),
            pltpu.SemaphoreType.DMA,
        ],
    )
    return kern(user_pairs, item_pairs, user_idx, item_idx)


def _mlp_body(gu_ref, gi_ref, pu_ref, pi_ref, w1_ref, b1_ref, w2_ref, b2_ref,
              o_ref):
    ones_row = jnp.ones((1, D), jnp.float32)
    dn0 = (((0,), (0,)), ((), ()))
    pu = lax.dot_general(pu_ref[...], ones_row, dn0,
                         preferred_element_type=jnp.float32)
    pi = lax.dot_general(pi_ref[...], ones_row, dn0,
                         preferred_element_type=jnp.float32)
    gu = gu_ref[...]
    gi = gi_ref[...]
    uv = gu[:, :D] + pu * (gu[:, D:] - gu[:, :D])
    iv = gi[:, :D] + pi * (gi[:, D:] - gi[:, :D])
    w1 = w1_ref[...]
    dn1 = (((1,), (1,)), ((), ()))
    h = lax.dot_general(uv, w1[:, :D], dn1,
                        preferred_element_type=jnp.float32,
                        precision=lax.Precision.HIGHEST)
    h = h + lax.dot_general(iv, w1[:, D:], dn1,
                            preferred_element_type=jnp.float32,
                            precision=lax.Precision.HIGHEST)
    h = jnp.maximum(h + b1_ref[...], 0.0)
    o = jnp.sum(h * w2_ref[...], axis=1, keepdims=True)
    o_ref[...] = jax.nn.sigmoid(o + b2_ref[0, 0])


def kernel(user_indices, item_indices, user_table, item_table, W1, b1, W2, b2):
    u32 = user_indices.astype(jnp.int32)
    i32 = item_indices.astype(jnp.int32)
    up = _pair_transpose(user_table.T)
    ip = _pair_transpose(item_table.T)
    u_hi = (u32 >= NPAIR).astype(jnp.int32)
    i_hi = (i32 >= NPAIR).astype(jnp.int32)
    gu, gi = _sc_gather(up, ip, u32 - u_hi * NPAIR, i32 - i_hi * NPAIR)
    pu = u_hi.astype(jnp.float32).reshape(1, B)
    pi = i_hi.astype(jnp.float32).reshape(1, B)
    blk = 1024
    out = pl.pallas_call(
        _mlp_body,
        grid=(B // blk,),
        in_specs=[
            pl.BlockSpec((blk, 2 * D), lambda i: (i, 0)),
            pl.BlockSpec((blk, 2 * D), lambda i: (i, 0)),
            pl.BlockSpec((1, blk), lambda i: (0, i)),
            pl.BlockSpec((1, blk), lambda i: (0, i)),
            pl.BlockSpec((H, 2 * D), lambda i: (0, 0)),
            pl.BlockSpec((1, H), lambda i: (0, 0)),
            pl.BlockSpec((1, H), lambda i: (0, 0)),
            pl.BlockSpec((1, 1), lambda i: (0, 0)),
        ],
        out_specs=pl.BlockSpec((blk, 1), lambda i: (i, 0)),
        out_shape=jax.ShapeDtypeStruct((B, 1), jnp.float32),
    )(gu, gi, pu, pi, W1, b1.reshape(1, H), W2, b2.reshape(1, 1))
    return out.reshape(B)


# speedup vs baseline: 2.2405x; 1.6920x over previous
import jax
import jax.numpy as jnp
from jax import lax
from jax.experimental import pallas as pl
from jax.experimental.pallas import tpu as pltpu
from jax.experimental.pallas import tpu_sc as plsc

B = 16384
D = 64
H = 64
NC = 2
NS = 16
NW = NC * NS
BPW = B // NW
CHUNK = 128
NCH = BPW // CHUNK

TBLK = 4096
NTS = 123
NPAIR = NTS * TBLK


def _transpose_body(lo_ref, hi_ref, o_ref):
    eye = jnp.eye(D, dtype=jnp.float32)
    dnT = (((0,), (0,)), ((), ()))
    o_ref[:, :D] = lax.dot_general(lo_ref[...], eye, dnT,
                                   preferred_element_type=jnp.float32)
    o_ref[:, D:] = lax.dot_general(hi_ref[...], eye, dnT,
                                   preferred_element_type=jnp.float32)


def _pair_transpose(tT):
    return pl.pallas_call(
        _transpose_body,
        grid=(NTS,),
        in_specs=[pl.BlockSpec((D, TBLK), lambda i: (0, i)),
                  pl.BlockSpec((D, TBLK),
                               lambda i: (0, jnp.minimum(i + NTS, 244)))],
        out_specs=pl.BlockSpec((TBLK, 2 * D), lambda i: (i, 0)),
        out_shape=jax.ShapeDtypeStruct((NPAIR, 2 * D), jnp.float32),
    )(tT, tT)


def _gather_body(u_tbl, i_tbl, u_idx, i_idx, u_out, i_out,
                 idx_v, rows_v, sem):
    wid = lax.axis_index("s") * NC + lax.axis_index("c")
    base = wid * BPW

    pltpu.sync_copy(u_idx.at[pl.ds(base, BPW)], idx_v)
    copies = [
        pltpu.async_copy(u_tbl.at[idx_v.at[pl.ds(c * CHUNK, CHUNK)]],
                         rows_v.at[pl.ds(c * CHUNK, CHUNK)], sem)
        for c in range(NCH)
    ]
    for cp in copies:
        cp.wait()
    pltpu.sync_copy(rows_v, u_out.at[pl.ds(base, BPW)])

    pltpu.sync_copy(i_idx.at[pl.ds(base, BPW)], idx_v)
    copies = [
        pltpu.async_copy(i_tbl.at[idx_v.at[pl.ds(c * CHUNK, CHUNK)]],
                         rows_v.at[pl.ds(c * CHUNK, CHUNK)], sem)
        for c in range(NCH)
    ]
    for cp in copies:
        cp.wait()
    pltpu.sync_copy(rows_v, i_out.at[pl.ds(base, BPW)])


def _sc_gather(user_pairs, item_pairs, user_idx, item_idx):
    mesh = plsc.VectorSubcoreMesh(core_axis_name="c", subcore_axis_name="s")
    kern = pl.kernel(
        _gather_body,
        out_type=[jax.ShapeDtypeStruct((B, 2 * D), jnp.float32),
                  jax.ShapeDtypeStruct((B, 2 * D), jnp.float32)],
        mesh=mesh,
        scratch_types=[
            pltpu.VMEM((BPW,), jnp.int32),
            pltpu.VMEM((BPW, 2 * D), jnp.float32),
            pltpu.SemaphoreType.DMA,
        ],
    )
    return kern(user_pairs, item_pairs, user_idx, item_idx)


def _mlp_body(gu_ref, gi_ref, pu_ref, pi_ref, w1_ref, b1_ref, w2_ref, b2_ref,
              o_ref):
    ones_row = jnp.ones((1, D), jnp.float32)
    dn0 = (((0,), (0,)), ((), ()))
    pu = lax.dot_general(pu_ref[...], ones_row, dn0,
                         preferred_element_type=jnp.float32)
    pi = lax.dot_general(pi_ref[...], ones_row, dn0,
                         preferred_element_type=jnp.float32)
    gu = gu_ref[...]
    gi = gi_ref[...]
    uv = gu[:, :D] + pu * (gu[:, D:] - gu[:, :D])
    iv = gi[:, :D] + pi * (gi[:, D:] - gi[:, :D])
    w1 = w1_ref[...]
    dn1 = (((1,), (1,)), ((), ()))
    h = lax.dot_general(uv, w1[:, :D], dn1,
                        preferred_element_type=jnp.float32,
                        precision=lax.Precision.HIGHEST)
    h = h + lax.dot_general(iv, w1[:, D:], dn1,
                            preferred_element_type=jnp.float32,
                            precision=lax.Precision.HIGHEST)
    h = jnp.maximum(h + b1_ref[...], 0.0)
    o = jnp.sum(h * w2_ref[...], axis=1, keepdims=True)
    o_ref[...] = jax.nn.sigmoid(o + b2_ref[0, 0])


def kernel(user_indices, item_indices, user_table, item_table, W1, b1, W2, b2):
    u32 = user_indices.astype(jnp.int32)
    i32 = item_indices.astype(jnp.int32)
    up = _pair_transpose(user_table.T)
    ip = _pair_transpose(item_table.T)
    u_hi = (u32 >= NPAIR).astype(jnp.int32)
    i_hi = (i32 >= NPAIR).astype(jnp.int32)
    gu, gi = _sc_gather(up, ip, u32 - u_hi * NPAIR, i32 - i_hi * NPAIR)
    pu = u_hi.astype(jnp.float32).reshape(1, B)
    pi = i_hi.astype(jnp.float32).reshape(1, B)
    blk = 1024
    out = pl.pallas_call(
        _mlp_body,
        grid=(B // blk,),
        in_specs=[
            pl.BlockSpec((blk, 2 * D), lambda i: (i, 0)),
            pl.BlockSpec((blk, 2 * D), lambda i: (i, 0)),
            pl.BlockSpec((1, blk), lambda i: (0, i)),
            pl.BlockSpec((1, blk), lambda i: (0, i)),
            pl.BlockSpec((H, 2 * D), lambda i: (0, 0)),
            pl.BlockSpec((1, H), lambda i: (0, 0)),
            pl.BlockSpec((1, H), lambda i: (0, 0)),
            pl.BlockSpec((1, 1), lambda i: (0, 0)),
        ],
        out_specs=pl.BlockSpec((blk, 1), lambda i: (i, 0)),
        out_shape=jax.ShapeDtypeStruct((B, 1), jnp.float32),
    )(gu, gi, pu, pi, W1, b1.reshape(1, H), W2, b2.reshape(1, 1))
    return out.reshape(B)


# speedup vs baseline: 2.5206x; 1.1250x over previous
import jax
import jax.numpy as jnp
from jax import lax
from jax.experimental import pallas as pl
from jax.experimental.pallas import tpu as pltpu
from jax.experimental.pallas import tpu_sc as plsc

B = 16384
D = 64
H = 64
NC = 2
NS = 16
NW = NC * NS
BPW = B // NW
CHUNK = 128
NCH = BPW // CHUNK

TBLK = 8192
NTS = 62
NPAIR = NTS * TBLK


def _transpose_body(lo_ref, hi_ref, o_ref):
    eye = jnp.eye(D, dtype=jnp.float32)
    dnT = (((0,), (0,)), ((), ()))
    o_ref[:, :D] = lax.dot_general(lo_ref[...], eye, dnT,
                                   preferred_element_type=jnp.float32)
    o_ref[:, D:] = lax.dot_general(hi_ref[...], eye, dnT,
                                   preferred_element_type=jnp.float32)


def _pair_transpose(tT):
    return pl.pallas_call(
        _transpose_body,
        grid=(NTS,),
        in_specs=[pl.BlockSpec((D, TBLK), lambda i: (0, i)),
                  pl.BlockSpec((D, TBLK),
                               lambda i: (0, jnp.minimum(i + NTS, 121)))],
        out_specs=pl.BlockSpec((TBLK, 2 * D), lambda i: (i, 0)),
        out_shape=jax.ShapeDtypeStruct((NPAIR, 2 * D), jnp.float32),
    )(tT, tT)


def _gather_body(u_tbl, i_tbl, u_idx, i_idx, u_out, i_out,
                 idx_v, rows_v, sem):
    wid = lax.axis_index("s") * NC + lax.axis_index("c")
    base = wid * BPW

    pltpu.sync_copy(u_idx.at[pl.ds(base, BPW)], idx_v)
    copies = [
        pltpu.async_copy(u_tbl.at[idx_v.at[pl.ds(c * CHUNK, CHUNK)]],
                         rows_v.at[pl.ds(c * CHUNK, CHUNK)], sem)
        for c in range(NCH)
    ]
    for cp in copies:
        cp.wait()
    pltpu.sync_copy(rows_v, u_out.at[pl.ds(base, BPW)])

    pltpu.sync_copy(i_idx.at[pl.ds(base, BPW)], idx_v)
    copies = [
        pltpu.async_copy(i_tbl.at[idx_v.at[pl.ds(c * CHUNK, CHUNK)]],
                         rows_v.at[pl.ds(c * CHUNK, CHUNK)], sem)
        for c in range(NCH)
    ]
    for cp in copies:
        cp.wait()
    pltpu.sync_copy(rows_v, i_out.at[pl.ds(base, BPW)])


def _sc_gather(user_pairs, item_pairs, user_idx, item_idx):
    mesh = plsc.VectorSubcoreMesh(core_axis_name="c", subcore_axis_name="s")
    kern = pl.kernel(
        _gather_body,
        out_type=[jax.ShapeDtypeStruct((B, 2 * D), jnp.float32),
                  jax.ShapeDtypeStruct((B, 2 * D), jnp.float32)],
        mesh=mesh,
        scratch_types=[
            pltpu.VMEM((BPW,), jnp.int32),
            pltpu.VMEM((BPW, 2 * D), jnp.float32),
            pltpu.SemaphoreType.DMA,
        ],
    )
    return kern(user_pairs, item_pairs, user_idx, item_idx)


def _mlp_body(gu_ref, gi_ref, pu_ref, pi_ref, w1_ref, b1_ref, w2_ref, b2_ref,
              o_ref):
    ones_row = jnp.ones((1, D), jnp.float32)
    dn0 = (((0,), (0,)), ((), ()))
    pu = lax.dot_general(pu_ref[...], ones_row, dn0,
                         preferred_element_type=jnp.float32)
    pi = lax.dot_general(pi_ref[...], ones_row, dn0,
                         preferred_element_type=jnp.float32)
    gu = gu_ref[...]
    gi = gi_ref[...]
    uv = gu[:, :D] + pu * (gu[:, D:] - gu[:, :D])
    iv = gi[:, :D] + pi * (gi[:, D:] - gi[:, :D])
    w1 = w1_ref[...]
    dn1 = (((1,), (1,)), ((), ()))
    h = lax.dot_general(uv, w1[:, :D], dn1,
                        preferred_element_type=jnp.float32,
                        precision=lax.Precision.HIGHEST)
    h = h + lax.dot_general(iv, w1[:, D:], dn1,
                            preferred_element_type=jnp.float32,
                            precision=lax.Precision.HIGHEST)
    h = jnp.maximum(h + b1_ref[...], 0.0)
    o = jnp.sum(h * w2_ref[...], axis=1, keepdims=True)
    o_ref[...] = jax.nn.sigmoid(o + b2_ref[0, 0])


def kernel(user_indices, item_indices, user_table, item_table, W1, b1, W2, b2):
    u32 = user_indices.astype(jnp.int32)
    i32 = item_indices.astype(jnp.int32)
    up = _pair_transpose(user_table.T)
    ip = _pair_transpose(item_table.T)
    u_hi = (u32 >= NPAIR).astype(jnp.int32)
    i_hi = (i32 >= NPAIR).astype(jnp.int32)
    gu, gi = _sc_gather(up, ip, u32 - u_hi * NPAIR, i32 - i_hi * NPAIR)
    pu = u_hi.astype(jnp.float32).reshape(1, B)
    pi = i_hi.astype(jnp.float32).reshape(1, B)
    blk = 1024
    out = pl.pallas_call(
        _mlp_body,
        grid=(B // blk,),
        in_specs=[
            pl.BlockSpec((blk, 2 * D), lambda i: (i, 0)),
            pl.BlockSpec((blk, 2 * D), lambda i: (i, 0)),
            pl.BlockSpec((1, blk), lambda i: (0, i)),
            pl.BlockSpec((1, blk), lambda i: (0, i)),
            pl.BlockSpec((H, 2 * D), lambda i: (0, 0)),
            pl.BlockSpec((1, H), lambda i: (0, 0)),
            pl.BlockSpec((1, H), lambda i: (0, 0)),
            pl.BlockSpec((1, 1), lambda i: (0, 0)),
        ],
        out_specs=pl.BlockSpec((blk, 1), lambda i: (i, 0)),
        out_shape=jax.ShapeDtypeStruct((B, 1), jnp.float32),
    )(gu, gi, pu, pi, W1, b1.reshape(1, H), W2, b2.reshape(1, 1))
    return out.reshape(B)
